# baseline (speedup 1.0000x reference)
import functools
import jax
import jax.numpy as jnp
from jax.experimental import pallas as pl

_H = 16
_DH = 64
_TOPK = 64
_TEMPERATURE = 1.0
_BQ = 256


def _qkv_body(x_ref, w_ref, b_ref, out_ref):
    x = x_ref[...]
    w = w_ref[0]
    b = b_ref[0]
    out_ref[0] = jnp.dot(x, w, preferred_element_type=jnp.float32) + b[0][None, :]


def _attn_body(q_ref, kt_ref, v_ref, o_ref, *, topk, scale):
    q = q_ref[0]
    kt = kt_ref[0]
    v = v_ref[0]
    s = jnp.dot(q, kt, preferred_element_type=jnp.float32) * scale

    def to_f32(k):
        return jax.lax.bitcast_convert_type(
            k ^ ((k >> 31) & jnp.int32(0x7FFFFFFF)), jnp.float32
        )

    bq = s.shape[0]
    lo0 = jnp.full((bq, 1), jnp.int32(-2139095041), jnp.int32)
    hi0 = jnp.full((bq, 1), jnp.int32(0x7F800000), jnp.int32)

    def avg(a, b):
        return (a >> 1) + (b >> 1) + (a & b & 1)

    def count_ge(thr):
        return jnp.sum(
            jnp.where(s >= to_f32(thr), jnp.int32(1), jnp.int32(0)),
            axis=1,
            keepdims=True,
        )

    lo, hi = lo0, hi0
    for _ in range(16):
        m2 = avg(lo, hi)
        m1 = avg(lo, m2)
        m3 = avg(m2, hi)
        c1 = count_ge(m1) >= topk
        c2 = count_ge(m2) >= topk
        c3 = count_ge(m3) >= topk
        new_lo = jnp.where(c3, m3, jnp.where(c2, m2, jnp.where(c1, m1, lo)))
        new_hi = jnp.where(c3, hi, jnp.where(c2, m3, jnp.where(c1, m2, m1)))
        lo, hi = new_lo, new_hi

    for _ in range(2):
        m2 = avg(lo, hi)
        c2 = count_ge(m2) >= topk
        lo = jnp.where(c2, m2, lo)
        hi = jnp.where(c2, hi, m2)

    t = to_f32(lo)
    m = jnp.max(s, axis=1, keepdims=True)
    w = jnp.where(s >= t, jnp.exp(s - m), 0.0)
    denom = jnp.sum(w, axis=1, keepdims=True)
    attn = w * (1.0 / denom)
    o_ref[0] = jnp.dot(attn, v, preferred_element_type=jnp.float32)


def _proj_body(x_ref, w_ref, b_ref, out_ref):
    out_ref[...] = (
        jnp.dot(x_ref[...], w_ref[...], preferred_element_type=jnp.float32)
        + b_ref[0][None, :]
    )


def kernel(x, Wq, bq, Wk, bk, Wv, bv, Wo, bo):
    b, s_len, d = x.shape
    h, dh = _H, d // _H
    scale = (dh ** -0.5) / _TEMPERATURE
    x2 = x.reshape(s_len, d)

    w3 = jnp.stack([Wq, Wk, Wv])
    b3 = jnp.stack([bq, bk, bv]).reshape(3, 1, d)

    nq = s_len // _BQ
    qkv = pl.pallas_call(
        _qkv_body,
        grid=(3, nq),
        in_specs=[
            pl.BlockSpec((_BQ, d), lambda j, i: (i, 0)),
            pl.BlockSpec((1, d, d), lambda j, i: (j, 0, 0)),
            pl.BlockSpec((1, 1, d), lambda j, i: (j, 0, 0)),
        ],
        out_specs=pl.BlockSpec((1, _BQ, d), lambda j, i: (j, i, 0)),
        out_shape=jax.ShapeDtypeStruct((3, s_len, d), jnp.float32),
    )(x2, w3, b3)

    q3 = qkv[0].reshape(s_len, h, dh).transpose(1, 0, 2)
    kt3 = qkv[1].reshape(s_len, h, dh).transpose(1, 2, 0)
    v3 = qkv[2].reshape(s_len, h, dh).transpose(1, 0, 2)

    o3 = pl.pallas_call(
        functools.partial(_attn_body, topk=_TOPK, scale=scale),
        grid=(h, nq),
        in_specs=[
            pl.BlockSpec((1, _BQ, dh), lambda hh, i: (hh, i, 0)),
            pl.BlockSpec((1, dh, s_len), lambda hh, i: (hh, 0, 0)),
            pl.BlockSpec((1, s_len, dh), lambda hh, i: (hh, 0, 0)),
        ],
        out_specs=pl.BlockSpec((1, _BQ, dh), lambda hh, i: (hh, i, 0)),
        out_shape=jax.ShapeDtypeStruct((h, s_len, dh), jnp.float32),
    )(q3, kt3, v3)

    o2 = o3.transpose(1, 0, 2).reshape(s_len, d)

    out = pl.pallas_call(
        _proj_body,
        grid=(nq,),
        in_specs=[
            pl.BlockSpec((_BQ, d), lambda i: (i, 0)),
            pl.BlockSpec((d, d), lambda i: (0, 0)),
            pl.BlockSpec((1, d), lambda i: (0, 0)),
        ],
        out_specs=pl.BlockSpec((_BQ, d), lambda i: (i, 0)),
        out_shape=jax.ShapeDtypeStruct((s_len, d), jnp.float32),
    )(o2, Wo, bo.reshape(1, d))

    return out.reshape(b, s_len, d)


# speedup vs baseline: 1.0004x; 1.0004x over previous
import functools
import jax
import jax.numpy as jnp
from jax import lax
from jax.experimental import pallas as pl
from jax.experimental.pallas import tpu as pltpu, tpu_sc as plsc

_H = 16
_TOPK = 64
_TEMPERATURE = 1.0
_BQ = 256
_NC = 2
_NS = 16
_NW = _NC * _NS


def _qkv_body(x_ref, w_ref, b_ref, out_ref):
    x = x_ref[...]
    out_ref[0] = (
        jnp.dot(x, w_ref[0], preferred_element_type=jnp.float32) + b_ref[0][0][None, :]
    )


def _keys_body(q_ref, kt_ref, keys_ref, *, scale):
    s = jnp.dot(q_ref[0], kt_ref[0], preferred_element_type=jnp.float32) * scale
    si = jax.lax.bitcast_convert_type(s, jnp.int32)
    keys_ref[0] = si ^ jnp.where(
        si < 0, jnp.int32(-1), jnp.int32(-2147483648)
    )


def _sc_body(keys_hbm, thr_hbm, batch_v, cand_v, hist_v, thr_v, *, rows, ss, topk, br):
    rw = rows // _NW
    nvr = ss // 16
    wid = lax.axis_index("s") * _NC + lax.axis_index("c")
    row0 = wid * rw
    iota = lax.iota(jnp.int32, 16)
    ones16 = jnp.ones((16,), jnp.int32)
    target = jnp.int32(ss - topk)

    def do_batch(b, _):
        base = row0 + b * br
        pltpu.sync_copy(keys_hbm.at[pl.ds(base, br)], batch_v)

        def do_row(rr, acc):
            for g in range(16):
                hist_v[pl.ds(g * 16, 16)] = jnp.zeros((16,), jnp.int32)

            def hist_step(j, _):
                x = batch_v[rr, pl.ds(j * 16, 16)]
                d = lax.shift_right_logical(x, 24)
                plsc.addupdate_scatter(hist_v, [d], ones16)
                return 0

            lax.fori_loop(0, nvr, hist_step, 0, unroll=4)

            def cum_step(g, carry):
                v = hist_v[pl.ds(g * 16, 16)]
                hist_v[pl.ds(g * 16, 16)] = plsc.cumsum(v) + carry
                return carry + jnp.sum(v)

            lax.fori_loop(0, 16, cum_step, jnp.int32(0))

            def find_step(g, carry):
                dmin, amin = carry
                a = hist_v[pl.ds(g * 16, 16)]
                over = a > target
                cand = jnp.where(over, iota + g * 16, jnp.int32(256))
                aval = jnp.where(over, a, jnp.int32(0x7FFFFFFF))
                return (
                    jnp.minimum(dmin, jnp.min(cand)),
                    jnp.minimum(amin, jnp.min(aval)),
                )

            dstar, a_dstar = lax.fori_loop(
                0, 16, find_step, (jnp.int32(256), jnp.int32(0x7FFFFFFF))
            )
            r2 = a_dstar - target

            def compact_step(j, nout):
                x = batch_v[rr, pl.ds(j * 16, 16)]
                d = lax.shift_right_logical(x, 24)
                msk = d == dstar
                mi = jnp.where(msk, jnp.int32(1), jnp.int32(0))
                pos = plsc.cumsum(mi) + (nout - 1)
                plsc.store_scatter(cand_v, [pos], x, mask=msk)
                return nout + jnp.sum(mi)

            nout = lax.fori_loop(0, nvr, compact_step, jnp.int32(0), unroll=4)

            nv2 = (nout + 15) >> 4
            mask24 = jnp.int32(0x00FFFFFF)

            def bis_step(_, lohi):
                lo, hi = lohi
                mid = (lo + hi) >> 1

                def cnt_step(j, c):
                    x = cand_v[pl.ds(j * 16, 16)]
                    valid = (iota + j * 16) < nout
                    return c + jnp.sum(
                        jnp.where(
                            valid & ((x & mask24) >= mid), jnp.int32(1), jnp.int32(0)
                        )
                    )

                cnt = lax.fori_loop(0, nv2, cnt_step, jnp.int32(0))
                ge = cnt >= r2
                return jnp.where(ge, mid, lo), jnp.where(ge, hi, mid)

            lo, _ = lax.fori_loop(
                0, 24, bis_step, (jnp.int32(0), jnp.int32(1 << 24))
            )
            thr = (dstar << 24) | lo
            return jnp.where(iota == rr, thr, acc)

        accf = lax.fori_loop(0, br, do_row, jnp.zeros((16,), jnp.int32))
        thr_v[pl.ds(b * br, 16)] = accf
        return 0

    lax.fori_loop(0, rw // br, do_batch, 0)
    pltpu.sync_copy(thr_v, thr_hbm.at[pl.ds(row0, rw)])


def _attn_body(q_ref, kt_ref, v_ref, thr_ref, o_ref, *, topk, scale):
    s = jnp.dot(q_ref[0], kt_ref[0], preferred_element_type=jnp.float32) * scale
    ti = thr_ref[0][0] ^ jnp.int32(-2147483648)
    t = jax.lax.bitcast_convert_type(
        ti ^ ((ti >> 31) & jnp.int32(0x7FFFFFFF)), jnp.float32
    )[:, None]
    m = jnp.max(s, axis=1, keepdims=True)
    w = jnp.where(s >= t, jnp.exp(s - m), 0.0)
    denom = jnp.sum(w, axis=1, keepdims=True)
    attn = w * (1.0 / denom)
    o_ref[0] = jnp.dot(attn, v_ref[0], preferred_element_type=jnp.float32)


def _proj_body(x_ref, w_ref, b_ref, out_ref):
    out_ref[...] = (
        jnp.dot(x_ref[...], w_ref[...], preferred_element_type=jnp.float32)
        + b_ref[0][None, :]
    )


def kernel(x, Wq, bq, Wk, bk, Wv, bv, Wo, bo):
    b, s_len, d = x.shape
    h, dh = _H, d // _H
    scale = (dh ** -0.5) / _TEMPERATURE
    x2 = x.reshape(s_len, d)

    w3 = jnp.stack([Wq, Wk, Wv])
    b3 = jnp.stack([bq, bk, bv]).reshape(3, 1, d)

    nq = s_len // _BQ
    qkv = pl.pallas_call(
        _qkv_body,
        grid=(3, nq),
        in_specs=[
            pl.BlockSpec((_BQ, d), lambda j, i: (i, 0)),
            pl.BlockSpec((1, d, d), lambda j, i: (j, 0, 0)),
            pl.BlockSpec((1, 1, d), lambda j, i: (j, 0, 0)),
        ],
        out_specs=pl.BlockSpec((1, _BQ, d), lambda j, i: (j, i, 0)),
        out_shape=jax.ShapeDtypeStruct((3, s_len, d), jnp.float32),
    )(x2, w3, b3)

    q3 = qkv[0].reshape(s_len, h, dh).transpose(1, 0, 2)
    kt3 = qkv[1].reshape(s_len, h, dh).transpose(1, 2, 0)
    v3 = qkv[2].reshape(s_len, h, dh).transpose(1, 0, 2)

    keys = pl.pallas_call(
        functools.partial(_keys_body, scale=scale),
        grid=(h, nq),
        in_specs=[
            pl.BlockSpec((1, _BQ, dh), lambda hh, i: (hh, i, 0)),
            pl.BlockSpec((1, dh, s_len), lambda hh, i: (hh, 0, 0)),
        ],
        out_specs=pl.BlockSpec((1, _BQ, s_len), lambda hh, i: (hh, i, 0)),
        out_shape=jax.ShapeDtypeStruct((h, s_len, s_len), jnp.int32),
    )(q3, kt3)

    rows = h * s_len
    keys2 = keys.reshape(rows, s_len)
    br = 16
    mesh = plsc.VectorSubcoreMesh(core_axis_name="c", subcore_axis_name="s")
    thr = pl.kernel(
        functools.partial(_sc_body, rows=rows, ss=s_len, topk=_TOPK, br=br),
        out_type=jax.ShapeDtypeStruct((rows,), jnp.int32),
        mesh=mesh,
        compiler_params=pltpu.CompilerParams(needs_layout_passes=False),
        scratch_types=[
            pltpu.VMEM((br, s_len), jnp.int32),
            pltpu.VMEM((s_len,), jnp.int32),
            pltpu.VMEM((256,), jnp.int32),
            pltpu.VMEM((rows // _NW,), jnp.int32),
        ],
    )(keys2)

    thr4 = thr.reshape(h * nq, 1, _BQ)

    o3 = pl.pallas_call(
        functools.partial(_attn_body, topk=_TOPK, scale=scale),
        grid=(h, nq),
        in_specs=[
            pl.BlockSpec((1, _BQ, dh), lambda hh, i: (hh, i, 0)),
            pl.BlockSpec((1, dh, s_len), lambda hh, i: (hh, 0, 0)),
            pl.BlockSpec((1, s_len, dh), lambda hh, i: (hh, 0, 0)),
            pl.BlockSpec((1, 1, _BQ), lambda hh, i: (hh * (s_len // _BQ) + i, 0, 0)),
        ],
        out_specs=pl.BlockSpec((1, _BQ, dh), lambda hh, i: (hh, i, 0)),
        out_shape=jax.ShapeDtypeStruct((h, s_len, dh), jnp.float32),
    )(q3, kt3, v3, thr4)

    o2 = o3.transpose(1, 0, 2).reshape(s_len, d)

    out = pl.pallas_call(
        _proj_body,
        grid=(nq,),
        in_specs=[
            pl.BlockSpec((_BQ, d), lambda i: (i, 0)),
            pl.BlockSpec((d, d), lambda i: (0, 0)),
            pl.BlockSpec((1, d), lambda i: (0, 0)),
        ],
        out_specs=pl.BlockSpec((_BQ, d), lambda i: (i, 0)),
        out_shape=jax.ShapeDtypeStruct((s_len, d), jnp.float32),
    )(o2, Wo, bo.reshape(1, d))

    return out.reshape(b, s_len, d)


# speedup vs baseline: 1.0701x; 1.0698x over previous
import functools
import jax
import jax.numpy as jnp
from jax import lax
from jax.experimental import pallas as pl
from jax.experimental.pallas import tpu as pltpu, tpu_sc as plsc

_H = 16
_TOPK = 64
_TEMPERATURE = 1.0
_BQ = 256
_NC = 2
_NS = 16
_NW = _NC * _NS


def _qkv_body(x_ref, w_ref, b_ref, out_ref):
    x = x_ref[...]
    out_ref[0] = (
        jnp.dot(x, w_ref[0], preferred_element_type=jnp.float32) + b_ref[0][0][None, :]
    )


def _keys_body(q_ref, kt_ref, keys_ref, *, scale):
    s = jnp.dot(q_ref[0], kt_ref[0], preferred_element_type=jnp.float32) * scale
    si = jax.lax.bitcast_convert_type(s, jnp.int32)
    keys_ref[0] = si ^ jnp.where(
        si < 0, jnp.int32(-1), jnp.int32(-2147483648)
    )


def _sc_body(keys_hbm, thr_hbm, batch_v, cand_v, hist_v, thr_v, *, rows, ss, topk, br):
    rw = rows // _NW
    nvr = ss // 16
    wid = lax.axis_index("s") * _NC + lax.axis_index("c")
    row0 = wid * rw
    iota = lax.iota(jnp.int32, 16)
    ones16 = jnp.ones((16,), jnp.int32)
    target = jnp.int32(ss - topk)

    def do_batch(b, _):
        base = row0 + b * br
        pltpu.sync_copy(keys_hbm.at[pl.ds(base, br)], batch_v)

        def do_row(rr, acc):
            for g in range(16):
                hist_v[pl.ds(g * 16, 16)] = jnp.zeros((16,), jnp.int32)

            def hist_step(j, _):
                x = batch_v[rr, pl.ds(j * 16, 16)]
                d = lax.shift_right_logical(x, 24)
                plsc.addupdate_scatter(hist_v, [d], ones16)
                return 0

            lax.fori_loop(0, nvr, hist_step, 0, unroll=4)

            def cum_step(g, carry):
                v = hist_v[pl.ds(g * 16, 16)]
                hist_v[pl.ds(g * 16, 16)] = plsc.cumsum(v) + carry
                return carry + jnp.sum(v)

            lax.fori_loop(0, 16, cum_step, jnp.int32(0))

            def find_step(g, carry):
                dmin, amin = carry
                a = hist_v[pl.ds(g * 16, 16)]
                over = a > target
                cand = jnp.where(over, iota + g * 16, jnp.int32(256))
                aval = jnp.where(over, a, jnp.int32(0x7FFFFFFF))
                return (
                    jnp.minimum(dmin, jnp.min(cand)),
                    jnp.minimum(amin, jnp.min(aval)),
                )

            dstar, a_dstar = lax.fori_loop(
                0, 16, find_step, (jnp.int32(256), jnp.int32(0x7FFFFFFF))
            )
            r2 = a_dstar - target

            def compact_step(j, nout_v):
                x = batch_v[rr, pl.ds(j * 16, 16)]
                d = lax.shift_right_logical(x, 24)
                msk = d == dstar
                mi = jnp.where(msk, jnp.int32(1), jnp.int32(0))
                pos = plsc.cumsum(mi) + (nout_v - 1)
                plsc.store_scatter(cand_v, [pos], x, mask=msk)
                return nout_v + plsc.all_reduce_population_count(msk)

            nout_v = lax.fori_loop(
                0, nvr, compact_step, jnp.zeros((16,), jnp.int32), unroll=4
            )
            nv2 = (jnp.max(nout_v) + 15) >> 4
            mask24 = jnp.int32(0x00FFFFFF)

            def bis_step(_, lohi):
                lo, hi = lohi
                mid = (lo + hi) >> 1

                def cnt_step(j, c):
                    x = cand_v[pl.ds(j * 16, 16)]
                    valid = (iota + j * 16) < nout_v
                    hit = valid & ((x & mask24) >= mid)
                    return c + plsc.all_reduce_population_count(hit)

                cnt = lax.fori_loop(0, nv2, cnt_step, jnp.zeros((16,), jnp.int32))
                ge = cnt >= r2
                return jnp.where(ge, mid, lo), jnp.where(ge, hi, mid)

            lo, _ = lax.fori_loop(
                0,
                24,
                bis_step,
                (jnp.zeros((16,), jnp.int32), jnp.full((16,), 1 << 24, jnp.int32)),
            )
            thr = (dstar << 24) | lo
            return jnp.where(iota == rr, thr, acc)

        accf = lax.fori_loop(0, br, do_row, jnp.zeros((16,), jnp.int32))
        thr_v[pl.ds(b * br, 16)] = accf
        return 0

    lax.fori_loop(0, rw // br, do_batch, 0)
    pltpu.sync_copy(thr_v, thr_hbm.at[pl.ds(row0, rw)])


def _attn_body(q_ref, kt_ref, v_ref, thr_ref, o_ref, *, topk, scale):
    s = jnp.dot(q_ref[0], kt_ref[0], preferred_element_type=jnp.float32) * scale
    ti = thr_ref[0][0] ^ jnp.int32(-2147483648)
    t = jax.lax.bitcast_convert_type(
        ti ^ ((ti >> 31) & jnp.int32(0x7FFFFFFF)), jnp.float32
    )[:, None]
    m = jnp.max(s, axis=1, keepdims=True)
    w = jnp.where(s >= t, jnp.exp(s - m), 0.0)
    denom = jnp.sum(w, axis=1, keepdims=True)
    attn = w * (1.0 / denom)
    o_ref[0] = jnp.dot(attn, v_ref[0], preferred_element_type=jnp.float32)


def _proj_body(x_ref, w_ref, b_ref, out_ref):
    out_ref[...] = (
        jnp.dot(x_ref[...], w_ref[...], preferred_element_type=jnp.float32)
        + b_ref[0][None, :]
    )


def kernel(x, Wq, bq, Wk, bk, Wv, bv, Wo, bo):
    b, s_len, d = x.shape
    h, dh = _H, d // _H
    scale = (dh ** -0.5) / _TEMPERATURE
    x2 = x.reshape(s_len, d)

    w3 = jnp.stack([Wq, Wk, Wv])
    b3 = jnp.stack([bq, bk, bv]).reshape(3, 1, d)

    nq = s_len // _BQ
    qkv = pl.pallas_call(
        _qkv_body,
        grid=(3, nq),
        in_specs=[
            pl.BlockSpec((_BQ, d), lambda j, i: (i, 0)),
            pl.BlockSpec((1, d, d), lambda j, i: (j, 0, 0)),
            pl.BlockSpec((1, 1, d), lambda j, i: (j, 0, 0)),
        ],
        out_specs=pl.BlockSpec((1, _BQ, d), lambda j, i: (j, i, 0)),
        out_shape=jax.ShapeDtypeStruct((3, s_len, d), jnp.float32),
    )(x2, w3, b3)

    q3 = qkv[0].reshape(s_len, h, dh).transpose(1, 0, 2)
    kt3 = qkv[1].reshape(s_len, h, dh).transpose(1, 2, 0)
    v3 = qkv[2].reshape(s_len, h, dh).transpose(1, 0, 2)

    keys = pl.pallas_call(
        functools.partial(_keys_body, scale=scale),
        grid=(h, nq),
        in_specs=[
            pl.BlockSpec((1, _BQ, dh), lambda hh, i: (hh, i, 0)),
            pl.BlockSpec((1, dh, s_len), lambda hh, i: (hh, 0, 0)),
        ],
        out_specs=pl.BlockSpec((1, _BQ, s_len), lambda hh, i: (hh, i, 0)),
        out_shape=jax.ShapeDtypeStruct((h, s_len, s_len), jnp.int32),
    )(q3, kt3)

    rows = h * s_len
    keys2 = keys.reshape(rows, s_len)
    br = 16
    mesh = plsc.VectorSubcoreMesh(core_axis_name="c", subcore_axis_name="s")
    thr = pl.kernel(
        functools.partial(_sc_body, rows=rows, ss=s_len, topk=_TOPK, br=br),
        out_type=jax.ShapeDtypeStruct((rows,), jnp.int32),
        mesh=mesh,
        compiler_params=pltpu.CompilerParams(needs_layout_passes=False),
        scratch_types=[
            pltpu.VMEM((br, s_len), jnp.int32),
            pltpu.VMEM((s_len,), jnp.int32),
            pltpu.VMEM((256,), jnp.int32),
            pltpu.VMEM((rows // _NW,), jnp.int32),
        ],
    )(keys2)

    thr4 = thr.reshape(h * nq, 1, _BQ)

    o3 = pl.pallas_call(
        functools.partial(_attn_body, topk=_TOPK, scale=scale),
        grid=(h, nq),
        in_specs=[
            pl.BlockSpec((1, _BQ, dh), lambda hh, i: (hh, i, 0)),
            pl.BlockSpec((1, dh, s_len), lambda hh, i: (hh, 0, 0)),
            pl.BlockSpec((1, s_len, dh), lambda hh, i: (hh, 0, 0)),
            pl.BlockSpec((1, 1, _BQ), lambda hh, i: (hh * (s_len // _BQ) + i, 0, 0)),
        ],
        out_specs=pl.BlockSpec((1, _BQ, dh), lambda hh, i: (hh, i, 0)),
        out_shape=jax.ShapeDtypeStruct((h, s_len, dh), jnp.float32),
    )(q3, kt3, v3, thr4)

    o2 = o3.transpose(1, 0, 2).reshape(s_len, d)

    out = pl.pallas_call(
        _proj_body,
        grid=(nq,),
        in_specs=[
            pl.BlockSpec((_BQ, d), lambda i: (i, 0)),
            pl.BlockSpec((d, d), lambda i: (0, 0)),
            pl.BlockSpec((1, d), lambda i: (0, 0)),
        ],
        out_specs=pl.BlockSpec((_BQ, d), lambda i: (i, 0)),
        out_shape=jax.ShapeDtypeStruct((s_len, d), jnp.float32),
    )(o2, Wo, bo.reshape(1, d))

    return out.reshape(b, s_len, d)
